# baseline (speedup 1.0000x reference)
import functools

import jax
import jax.numpy as jnp
from jax import lax
from jax.experimental import pallas as pl
from jax.experimental.pallas import tpu as pltpu

N, E, D, A, H, NG = 10000, 320000, 128, 64, 256, 64


def _dot_t(x, w):
    return lax.dot_general(x, w, (((1,), (1,)), ((), ())),
                           preferred_element_type=jnp.float32)


def _ln_rows(x, w, b):
    mu = jnp.mean(x, axis=-1, keepdims=True)
    var = jnp.mean((x - mu) ** 2, axis=-1, keepdims=True)
    return (x - mu) * lax.rsqrt(var + 1e-5) * w + b


def _stage_a_body(x_ref, bidx_row_ref, bidx_col_ref, gtok_ref, wgin_ref,
                  wginb_ref, n1w_ref, n1b_ref, msgw_ref, msgb_ref,
                  x1_ref, xn1_ref, m_ref):
    X = x_ref[:, :]
    bidx_row = bidx_row_ref[:, :]
    bidx_col = bidx_col_ref[:, :]
    iota_g = lax.broadcasted_iota(jnp.int32, (NG, N), 0)
    onehot = (bidx_row == iota_g).astype(jnp.float32)
    iota_n = lax.broadcasted_iota(jnp.int32, (N, NG), 1)
    onehot_t = (bidx_col == iota_n).astype(jnp.float32)
    sums = jnp.dot(onehot, X, preferred_element_type=jnp.float32)
    counts = jnp.maximum(jnp.sum(onehot, axis=1, keepdims=True), 1.0)
    g = gtok_ref[:, :] + sums / counts
    xg = _dot_t(g, wgin_ref[:, :]) + wginb_ref[:, :]
    x1 = X + jnp.dot(onehot_t, xg, preferred_element_type=jnp.float32)
    xn1 = _ln_rows(x1, n1w_ref[:, :], n1b_ref[:, :])
    x1_ref[:, :] = x1
    xn1_ref[:, :] = xn1
    m_ref[:, :] = _dot_t(xn1, msgw_ref[:, :]) + msgb_ref[:, :]


def _stage_a(X, batch_idx, g_token, Wg_in_W, Wg_in_b, norm1_w, norm1_b,
             local_msg_W, local_msg_b):
    out_shapes = (
        jax.ShapeDtypeStruct((N, D), jnp.float32),
        jax.ShapeDtypeStruct((N, D), jnp.float32),
        jax.ShapeDtypeStruct((N, D), jnp.float32),
    )
    return pl.pallas_call(
        _stage_a_body,
        out_shape=out_shapes,
    )(X, batch_idx.reshape(1, N), batch_idx.reshape(N, 1),
      g_token.reshape(1, D), Wg_in_W, Wg_in_b.reshape(1, D),
      norm1_w.reshape(1, D), norm1_b.reshape(1, D),
      local_msg_W, local_msg_b.reshape(1, D))


def _stage_b_body(x1_ref, xn1_ref, agg_ref, eps_ref, linw_ref, linb_ref,
                  n2w_ref, n2b_ref, wq_ref, wk_ref, wy_ref, wyb_ref,
                  x2_ref, q_ref, k_ref, kv_ref):
    xn1 = xn1_ref[:, :]
    pre = (1.0 + eps_ref[0, 0]) * xn1 + agg_ref[:, :]
    y = jnp.maximum(_dot_t(pre, linw_ref[:, :]) + linb_ref[:, :], 0.0)
    x2 = x1_ref[:, :] + y
    xn2 = _ln_rows(x2, n2w_ref[:, :], n2b_ref[:, :])
    q = _dot_t(xn2, wq_ref[:, :])
    k = _dot_t(xn2, wk_ref[:, :])
    vy = _dot_t(xn2, wy_ref[:, :]) + wyb_ref[:, :]
    x2_ref[:, :] = x2
    q_ref[:, :] = q
    k_ref[:, :] = k
    kv_ref[:, 0:A] = k
    kv_ref[:, A:A + D] = vy


def _stage_b(X1, Xn1, agg, local_eps, local_lin_W, local_lin_b,
             norm2_w, norm2_b, Wq, Wk, Wy_W, Wy_b):
    out_shapes = (
        jax.ShapeDtypeStruct((N, D), jnp.float32),
        jax.ShapeDtypeStruct((N, A), jnp.float32),
        jax.ShapeDtypeStruct((N, A), jnp.float32),
        jax.ShapeDtypeStruct((N, A + D), jnp.float32),
    )
    return pl.pallas_call(
        _stage_b_body,
        out_shape=out_shapes,
    )(X1, Xn1, agg, local_eps.reshape(1, 1), local_lin_W,
      local_lin_b.reshape(1, D), norm2_w.reshape(1, D),
      norm2_b.reshape(1, D), Wq, Wk, Wy_W, Wy_b.reshape(1, D))


def _stage_c_body(x2_ref, p_ref, u_ref, s_ref, wew_ref, web_ref,
                  n3w_ref, n3b_ref, w1_ref, b1_ref, w2_ref, b2_ref,
                  out_ref):
    y2 = u_ref[:, :] + _dot_t(p_ref[:, :], wew_ref[:, :]) \
        + s_ref[:, :] * web_ref[:, :]
    x3 = x2_ref[:, :] + y2
    xn3 = _ln_rows(x3, n3w_ref[:, :], n3b_ref[:, :])
    h = jnp.maximum(_dot_t(xn3, w1_ref[:, :]) + b1_ref[:, :], 0.0)
    out_ref[:, :] = x3 + _dot_t(h, w2_ref[:, :]) + b2_ref[:, :]


def _stage_c(X2, P, U, S, We_W, We_b, norm3_w, norm3_b,
             ffn1_W, ffn1_b, ffn2_W, ffn2_b):
    return pl.pallas_call(
        _stage_c_body,
        out_shape=jax.ShapeDtypeStruct((N, D), jnp.float32),
    )(X2, P, U, S.reshape(N, 1), We_W, We_b.reshape(1, D),
      norm3_w.reshape(1, D), norm3_b.reshape(1, D),
      ffn1_W, ffn1_b.reshape(1, H), ffn2_W, ffn2_b.reshape(1, D))


def kernel(X, edge_index, batch_idx, g_token, Wg_in_W, Wg_in_b, norm1_w,
           norm1_b, local_eps, local_msg_W, local_msg_b, local_lin_W,
           local_lin_b, norm2_w, norm2_b, Wq, Wk, Wphi, Wy_W, Wy_b, We_W,
           We_b, norm3_w, norm3_b, ffn1_W, ffn1_b, ffn2_W, ffn2_b,
           Wg_out_W, Wg_out_b):
    src, dst = edge_index[0], edge_index[1]

    X1, Xn1, M = _stage_a(X, batch_idx, g_token, Wg_in_W, Wg_in_b,
                          norm1_w, norm1_b, local_msg_W, local_msg_b)

    agg = jnp.zeros((N, D), jnp.float32).at[src].add(M[dst])

    X2, Q, K, KV = _stage_b(X1, Xn1, agg, local_eps, local_lin_W,
                            local_lin_b, norm2_w, norm2_b, Wq, Wk,
                            Wy_W, Wy_b)

    shift = jnp.sum(jnp.abs(Wphi))
    e = jnp.tanh(Q[src] + K[dst])
    scores = e @ Wphi[0] - shift
    exp_scores = jnp.exp(scores)
    denom = jnp.zeros((N,), jnp.float32).at[src].add(exp_scores)
    r = 1.0 / (denom + 1e-9)
    alpha = exp_scores * r[src]
    P = jnp.zeros((N, A), jnp.float32).at[src].add(alpha[:, None] * e)
    U = jnp.zeros((N, D), jnp.float32).at[src].add(
        alpha[:, None] * KV[dst, A:A + D])
    S = denom * r

    return _stage_c(X2, P, U, S, We_W, We_b, norm3_w, norm3_b,
                    ffn1_W, ffn1_b, ffn2_W, ffn2_b)


# speedup vs baseline: 107.3844x; 107.3844x over previous
import functools

import jax
import jax.numpy as jnp
from jax import lax
from jax.experimental import pallas as pl
from jax.experimental.pallas import tpu as pltpu
from jax.experimental.pallas import tpu_sc as plsc

N, E, D, A, H, NG = 10000, 320000, 128, 64, 256, 64

NC, NS = 2, 16
NW = NC * NS
EPW = E // NW
C = 80
NCHUNK = EPW // C
NXS = 10
NPS = N // NXS
ZB = 40
NZB = NPS // ZB

_sc_mesh = plsc.VectorSubcoreMesh(core_axis_name="c", subcore_axis_name="s")


def _dot_t(x, w):
    return lax.dot_general(x, w, (((1,), (1,)), ((), ())),
                           preferred_element_type=jnp.float32)


def _ln_rows(x, w, b):
    mu = jnp.mean(x, axis=-1, keepdims=True)
    var = jnp.mean((x - mu) ** 2, axis=-1, keepdims=True)
    return (x - mu) * lax.rsqrt(var + 1e-5) * w + b


def _tanh(x):
    t = jnp.exp(x * 2.0)
    return 1.0 - 2.0 / (t + 1.0)


def _allsum16(x):
    dnums = lax.GatherDimensionNumbers(
        offset_dims=(), collapsed_slice_dims=(0,), start_index_map=(0,))
    lanes = lax.broadcasted_iota(jnp.int32, (16,), 0)
    for k in (1, 2, 4, 8):
        perm = (lanes ^ k).reshape(16, 1)
        x = x + lax.gather(x, perm, dnums, slice_sizes=(1,),
                           mode=lax.GatherScatterMode.PROMISE_IN_BOUNDS)
    return x


def _zero_bounce(zb_v):
    def zrow(z, zcarry):
        for j in range(D // 16):
            zb_v[z, pl.ds(j * 16, 16)] = jnp.zeros((16,), jnp.float32)
        return zcarry

    lax.fori_loop(0, ZB, zrow, 0)


def _init_acc(sid, zb_v, acc_sh):
    def zcp(t, zcarry):
        pltpu.sync_copy(zb_v, acc_sh.at[pl.ds(sid * NPS + t * ZB, ZB)])
        return zcarry

    lax.fori_loop(0, NZB, zcp, 0)


def _export_acc(cid, sid, zb_v, acc_sh, out_hbm):
    def ecp(t, ecarry):
        roff = sid * NPS + t * ZB
        pltpu.sync_copy(acc_sh.at[pl.ds(roff, ZB)], zb_v)
        pltpu.sync_copy(zb_v, out_hbm.at[pl.ds(cid * N + roff, ZB)])
        return ecarry

    lax.fori_loop(0, NZB, ecp, 0)


def _stage_a_body(x_ref, bidx_row_ref, bidx_col_ref, gtok_ref, wgin_ref,
                  wginb_ref, n1w_ref, n1b_ref, msgw_ref, msgb_ref,
                  x1_ref, xn1_ref, m_ref):
    X = x_ref[:, :]
    iota_g = lax.broadcasted_iota(jnp.int32, (NG, N), 0)
    onehot = (bidx_row_ref[:, :] == iota_g).astype(jnp.float32)
    iota_n = lax.broadcasted_iota(jnp.int32, (N, NG), 1)
    onehot_t = (bidx_col_ref[:, :] == iota_n).astype(jnp.float32)
    sums = jnp.dot(onehot, X, preferred_element_type=jnp.float32)
    counts = jnp.maximum(jnp.sum(onehot, axis=1, keepdims=True), 1.0)
    g = gtok_ref[:, :] + sums / counts
    xg = _dot_t(g, wgin_ref[:, :]) + wginb_ref[:, :]
    x1 = X + jnp.dot(onehot_t, xg, preferred_element_type=jnp.float32)
    xn1 = _ln_rows(x1, n1w_ref[:, :], n1b_ref[:, :])
    x1_ref[:, :] = x1
    xn1_ref[:, :] = xn1
    m_ref[:, :] = _dot_t(xn1, msgw_ref[:, :]) + msgb_ref[:, :]


def _stage_a(X, batch_idx, g_token, Wg_in_W, Wg_in_b, norm1_w, norm1_b,
             local_msg_W, local_msg_b):
    out_shapes = (
        jax.ShapeDtypeStruct((N, D), jnp.float32),
        jax.ShapeDtypeStruct((N, D), jnp.float32),
        jax.ShapeDtypeStruct((N, D), jnp.float32),
    )
    return pl.pallas_call(_stage_a_body, out_shape=out_shapes)(
        X, batch_idx.reshape(1, N), batch_idx.reshape(N, 1),
        g_token.reshape(1, D), Wg_in_W, Wg_in_b.reshape(1, D),
        norm1_w.reshape(1, D), norm1_b.reshape(1, D),
        local_msg_W, local_msg_b.reshape(1, D))


@functools.partial(
    pl.kernel,
    out_type=jax.ShapeDtypeStruct((NC * N, D), jnp.float32),
    mesh=_sc_mesh,
    scratch_types=[
        pltpu.VMEM((C,), jnp.int32),
        pltpu.VMEM((C,), jnp.int32),
        pltpu.VMEM((C, D), jnp.float32),
        pltpu.VMEM((ZB, D), jnp.float32),
        pltpu.VMEM_SHARED((N, D), jnp.float32),
        pltpu.SemaphoreType.DMA,
    ],
    name="sc_mpnn_agg",
)
def _sc_agg(src_hbm, dst_hbm, m_hbm, out_hbm,
            src_v, dst_v, rows_v, zb_v, acc_sh, sem):
    cid = lax.axis_index("c")
    sid = lax.axis_index("s")
    base = (sid * NC + cid) * EPW
    _zero_bounce(zb_v)

    @pl.when(sid < NXS)
    def _():
        _init_acc(sid, zb_v, acc_sh)

    plsc.subcore_barrier()

    def chunk(i, carry):
        off = base + i * C
        pltpu.sync_copy(src_hbm.at[pl.ds(off, C)], src_v)
        pltpu.sync_copy(dst_hbm.at[pl.ds(off, C)], dst_v)
        pltpu.async_copy(m_hbm.at[dst_v], rows_v, sem).wait()
        pltpu.sync_copy(rows_v, acc_sh.at[src_v], add=True)
        return carry

    lax.fori_loop(0, NCHUNK, chunk, 0)
    plsc.subcore_barrier()

    @pl.when(sid < NXS)
    def _():
        _export_acc(cid, sid, zb_v, acc_sh, out_hbm)


def _stage_b_body(x1_ref, xn1_ref, agg0_ref, agg1_ref, eps_ref, linw_ref,
                  linb_ref, n2w_ref, n2b_ref, wq_ref, wk_ref, wy_ref,
                  wyb_ref, wphi_ref, x2_ref, qk_ref, vy_ref, wext_ref):
    xn1 = xn1_ref[:, :]
    agg = agg0_ref[:, :] + agg1_ref[:, :]
    pre = (1.0 + eps_ref[0, 0]) * xn1 + agg
    y = jnp.maximum(_dot_t(pre, linw_ref[:, :]) + linb_ref[:, :], 0.0)
    x2 = x1_ref[:, :] + y
    xn2 = _ln_rows(x2, n2w_ref[:, :], n2b_ref[:, :])
    qk_ref[:, 0:A] = _dot_t(xn2, wq_ref[:, :])
    qk_ref[:, A:2 * A] = _dot_t(xn2, wk_ref[:, :])
    vy_ref[:, :] = _dot_t(xn2, wy_ref[:, :]) + wyb_ref[:, :]
    x2_ref[:, :] = x2
    wphi = wphi_ref[:, :]
    wext_ref[:, :] = jnp.zeros((1, 80), jnp.float32)
    wext_ref[:, 0:A] = wphi
    wext_ref[:, A:A + 1] = jnp.sum(jnp.abs(wphi), axis=1, keepdims=True)


def _stage_b(X1, Xn1, aggp, local_eps, local_lin_W, local_lin_b,
             norm2_w, norm2_b, Wq, Wk, Wphi, Wy_W, Wy_b):
    out_shapes = (
        jax.ShapeDtypeStruct((N, D), jnp.float32),
        jax.ShapeDtypeStruct((N, 2 * A), jnp.float32),
        jax.ShapeDtypeStruct((N, D), jnp.float32),
        jax.ShapeDtypeStruct((1, 80), jnp.float32),
    )
    return pl.pallas_call(_stage_b_body, out_shape=out_shapes)(
        X1, Xn1, aggp[:N], aggp[N:], local_eps.reshape(1, 1),
        local_lin_W, local_lin_b.reshape(1, D), norm2_w.reshape(1, D),
        norm2_b.reshape(1, D), Wq, Wk, Wy_W, Wy_b.reshape(1, D), Wphi)


@functools.partial(
    pl.kernel,
    out_type=jax.ShapeDtypeStruct((NC * N, D), jnp.float32),
    mesh=_sc_mesh,
    scratch_types=[
        pltpu.VMEM((C,), jnp.int32),
        pltpu.VMEM((C,), jnp.int32),
        pltpu.VMEM((C, 2 * A), jnp.float32),
        pltpu.VMEM((C, 2 * A), jnp.float32),
        pltpu.VMEM((C, D), jnp.float32),
        pltpu.VMEM((ZB, D), jnp.float32),
        pltpu.VMEM((80,), jnp.float32),
        pltpu.VMEM((8, D), jnp.float32),
        pltpu.VMEM_SHARED((N, D), jnp.float32),
        pltpu.SemaphoreType.DMA,
    ],
    name="sc_attn_u",
)
def _sc_attu(src_hbm, dst_hbm, qk_hbm, vy_hbm, wext_hbm, dep_hbm, out_hbm,
             src_v, dst_v, qs_v, kd_v, vy_v, zb_v, w_v, dep_v, acc_sh,
             sem):
    cid = lax.axis_index("c")
    sid = lax.axis_index("s")
    base = (sid * NC + cid) * EPW
    pltpu.sync_copy(wext_hbm, w_v)
    pltpu.sync_copy(dep_hbm.at[pl.ds(0, 8)], dep_v)
    _zero_bounce(zb_v)

    @pl.when(sid < NXS)
    def _():
        _init_acc(sid, zb_v, acc_sh)

    plsc.subcore_barrier()
    wphi = [w_v[pl.ds(j * 16, 16)] for j in range(A // 16)]
    shift = w_v[pl.ds(A, 16)][0]

    def chunk(i, carry):
        off = base + i * C
        pltpu.sync_copy(src_hbm.at[pl.ds(off, C)], src_v)
        pltpu.sync_copy(dst_hbm.at[pl.ds(off, C)], dst_v)
        cp_q = pltpu.async_copy(qk_hbm.at[src_v], qs_v, sem)
        cp_k = pltpu.async_copy(qk_hbm.at[dst_v], kd_v, sem)
        cp_v = pltpu.async_copy(vy_hbm.at[dst_v], vy_v, sem)
        cp_q.wait()
        cp_k.wait()
        cp_v.wait()

        def edge(c, ecarry):
            acc = jnp.zeros((16,), jnp.float32)
            for j in range(A // 16):
                e = _tanh(qs_v[c, pl.ds(j * 16, 16)]
                          + kd_v[c, pl.ds(A + j * 16, 16)])
                acc = acc + e * wphi[j]
            esv = jnp.exp(_allsum16(acc) - shift)
            for j in range(D // 16):
                sl = pl.ds(j * 16, 16)
                vy_v[c, sl] = esv * vy_v[c, sl]
            return ecarry

        lax.fori_loop(0, C, edge, 0)
        pltpu.sync_copy(vy_v, acc_sh.at[src_v], add=True)
        return carry

    lax.fori_loop(0, NCHUNK, chunk, 0)
    plsc.subcore_barrier()

    @pl.when(sid < NXS)
    def _():
        _export_acc(cid, sid, zb_v, acc_sh, out_hbm)


@functools.partial(
    pl.kernel,
    out_type=jax.ShapeDtypeStruct((NC * N, D), jnp.float32),
    mesh=_sc_mesh,
    scratch_types=[
        pltpu.VMEM((C,), jnp.int32),
        pltpu.VMEM((C,), jnp.int32),
        pltpu.VMEM((C, 2 * A), jnp.float32),
        pltpu.VMEM((C, 2 * A), jnp.float32),
        pltpu.VMEM((C, D), jnp.float32),
        pltpu.VMEM((ZB, D), jnp.float32),
        pltpu.VMEM((80,), jnp.float32),
        pltpu.VMEM((8, D), jnp.float32),
        pltpu.VMEM_SHARED((N, D), jnp.float32),
        pltpu.SemaphoreType.DMA,
    ],
    name="sc_attn_pe",
)
def _sc_attpe(src_hbm, dst_hbm, qk_hbm, wext_hbm, dep_hbm, out_hbm,
              src_v, dst_v, qs_v, kd_v, pe_v, zb_v, w_v, dep_v, acc_sh,
              sem):
    cid = lax.axis_index("c")
    sid = lax.axis_index("s")
    base = (sid * NC + cid) * EPW
    pltpu.sync_copy(wext_hbm, w_v)
    pltpu.sync_copy(dep_hbm.at[pl.ds(0, 8)], dep_v)
    _zero_bounce(zb_v)

    @pl.when(sid < NXS)
    def _():
        _init_acc(sid, zb_v, acc_sh)

    plsc.subcore_barrier()
    wphi = [w_v[pl.ds(j * 16, 16)] for j in range(A // 16)]
    shift = w_v[pl.ds(A, 16)][0]
    lane0 = lax.broadcasted_iota(jnp.int32, (16,), 0) == 0
    zeros16 = jnp.zeros((16,), jnp.float32)

    def chunk(i, carry):
        off = base + i * C
        pltpu.sync_copy(src_hbm.at[pl.ds(off, C)], src_v)
        pltpu.sync_copy(dst_hbm.at[pl.ds(off, C)], dst_v)
        cp_q = pltpu.async_copy(qk_hbm.at[src_v], qs_v, sem)
        cp_k = pltpu.async_copy(qk_hbm.at[dst_v], kd_v, sem)
        cp_q.wait()
        cp_k.wait()

        def edge(c, ecarry):
            acc = jnp.zeros((16,), jnp.float32)
            es = []
            for j in range(A // 16):
                e = _tanh(qs_v[c, pl.ds(j * 16, 16)]
                          + kd_v[c, pl.ds(A + j * 16, 16)])
                es.append(e)
                acc = acc + e * wphi[j]
            esv = jnp.exp(_allsum16(acc) - shift)
            for j in range(A // 16):
                pe_v[c, pl.ds(j * 16, 16)] = esv * es[j]
            pe_v[c, pl.ds(A, 16)] = jnp.where(lane0, esv, 0.0)
            pe_v[c, pl.ds(A + 16, 16)] = zeros16
            pe_v[c, pl.ds(A + 32, 16)] = zeros16
            pe_v[c, pl.ds(A + 48, 16)] = zeros16
            return ecarry

        lax.fori_loop(0, C, edge, 0)
        pltpu.sync_copy(pe_v, acc_sh.at[src_v], add=True)
        return carry

    lax.fori_loop(0, NCHUNK, chunk, 0)
    plsc.subcore_barrier()

    @pl.when(sid < NXS)
    def _():
        _export_acc(cid, sid, zb_v, acc_sh, out_hbm)


def _stage_c_body(x2_ref, u0_ref, u1_ref, pe0_ref, pe1_ref,
                  wew_ref, web_ref, n3w_ref, n3b_ref, w1_ref, b1_ref,
                  w2_ref, b2_ref, out_ref):
    pe = pe0_ref[:, :] + pe1_ref[:, :]
    denom = pe[:, A:A + 1]
    r = 1.0 / (denom + 1e-9)
    p = pe[:, 0:A] * r
    u = (u0_ref[:, :] + u1_ref[:, :]) * r
    s = denom * r
    y2 = u + _dot_t(p, wew_ref[:, :]) + s * web_ref[:, :]
    x3 = x2_ref[:, :] + y2
    xn3 = _ln_rows(x3, n3w_ref[:, :], n3b_ref[:, :])
    h = jnp.maximum(_dot_t(xn3, w1_ref[:, :]) + b1_ref[:, :], 0.0)
    out_ref[:, :] = x3 + _dot_t(h, w2_ref[:, :]) + b2_ref[:, :]


def _stage_c(X2, Up, PEp, We_W, We_b, norm3_w, norm3_b,
             ffn1_W, ffn1_b, ffn2_W, ffn2_b):
    return pl.pallas_call(
        _stage_c_body,
        out_shape=jax.ShapeDtypeStruct((N, D), jnp.float32),
    )(X2, Up[:N], Up[N:], PEp[:N], PEp[N:], We_W, We_b.reshape(1, D),
      norm3_w.reshape(1, D), norm3_b.reshape(1, D),
      ffn1_W, ffn1_b.reshape(1, H), ffn2_W, ffn2_b.reshape(1, D))


def kernel(X, edge_index, batch_idx, g_token, Wg_in_W, Wg_in_b, norm1_w,
           norm1_b, local_eps, local_msg_W, local_msg_b, local_lin_W,
           local_lin_b, norm2_w, norm2_b, Wq, Wk, Wphi, Wy_W, Wy_b, We_W,
           We_b, norm3_w, norm3_b, ffn1_W, ffn1_b, ffn2_W, ffn2_b,
           Wg_out_W, Wg_out_b):
    src, dst = edge_index[0], edge_index[1]

    X1, Xn1, M = _stage_a(X, batch_idx, g_token, Wg_in_W, Wg_in_b,
                          norm1_w, norm1_b, local_msg_W, local_msg_b)
    aggp = _sc_agg(src, dst, M)
    X2, QK, VY, wext = _stage_b(X1, Xn1, aggp, local_eps, local_lin_W,
                                local_lin_b, norm2_w, norm2_b, Wq, Wk,
                                Wphi, Wy_W, Wy_b)
    wext = wext.reshape(80)
    Up = _sc_attu(src, dst, QK, VY, wext, aggp)
    PEp = _sc_attpe(src, dst, QK, wext, Up)
    return _stage_c(X2, Up, PEp, We_W, We_b, norm3_w, norm3_b,
                    ffn1_W, ffn1_b, ffn2_W, ffn2_b)


# speedup vs baseline: 117.3762x; 1.0930x over previous
import functools

import jax
import jax.numpy as jnp
from jax import lax
from jax.experimental import pallas as pl
from jax.experimental.pallas import tpu as pltpu
from jax.experimental.pallas import tpu_sc as plsc

N, E, D, A, H, NG = 10000, 320000, 128, 64, 256, 64

NC, NS = 2, 16
NW = NC * NS
EPW = E // NW
C = 80
NCHUNK = EPW // C
NXS = 10
NPS = N // NXS
ZB = 40
NZB = NPS // ZB

_sc_mesh = plsc.VectorSubcoreMesh(core_axis_name="c", subcore_axis_name="s")


def _dot_t(x, w):
    return lax.dot_general(x, w, (((1,), (1,)), ((), ())),
                           preferred_element_type=jnp.float32)


def _ln_rows(x, w, b):
    mu = jnp.mean(x, axis=-1, keepdims=True)
    var = jnp.mean((x - mu) ** 2, axis=-1, keepdims=True)
    return (x - mu) * lax.rsqrt(var + 1e-5) * w + b


def _tanh(x):
    t = jnp.exp(x * 2.0)
    return 1.0 - 2.0 / (t + 1.0)


def _allsum16(x):
    dnums = lax.GatherDimensionNumbers(
        offset_dims=(), collapsed_slice_dims=(0,), start_index_map=(0,))
    lanes = lax.broadcasted_iota(jnp.int32, (16,), 0)
    for k in (1, 2, 4, 8):
        perm = (lanes ^ k).reshape(16, 1)
        x = x + lax.gather(x, perm, dnums, slice_sizes=(1,),
                           mode=lax.GatherScatterMode.PROMISE_IN_BOUNDS)
    return x


def _zero_bounce(zb_v):
    def zrow(z, zcarry):
        for j in range(D // 16):
            zb_v[z, pl.ds(j * 16, 16)] = jnp.zeros((16,), jnp.float32)
        return zcarry

    lax.fori_loop(0, ZB, zrow, 0)


def _init_acc(sid, zb_v, acc_sh):
    def zcp(t, zcarry):
        pltpu.sync_copy(zb_v, acc_sh.at[pl.ds(sid * NPS + t * ZB, ZB)])
        return zcarry

    lax.fori_loop(0, NZB, zcp, 0)


def _export_acc(cid, sid, zb_v, acc_sh, out_hbm):
    def ecp(t, ecarry):
        roff = sid * NPS + t * ZB
        pltpu.sync_copy(acc_sh.at[pl.ds(roff, ZB)], zb_v)
        pltpu.sync_copy(zb_v, out_hbm.at[pl.ds(cid * N + roff, ZB)])
        return ecarry

    lax.fori_loop(0, NZB, ecp, 0)


def _stage_a_body(x_ref, bidx_row_ref, bidx_col_ref, gtok_ref, wgin_ref,
                  wginb_ref, n1w_ref, n1b_ref, msgw_ref, msgb_ref,
                  x1_ref, xn1_ref, m_ref):
    X = x_ref[:, :]
    iota_g = lax.broadcasted_iota(jnp.int32, (NG, N), 0)
    onehot = (bidx_row_ref[:, :] == iota_g).astype(jnp.float32)
    iota_n = lax.broadcasted_iota(jnp.int32, (N, NG), 1)
    onehot_t = (bidx_col_ref[:, :] == iota_n).astype(jnp.float32)
    sums = jnp.dot(onehot, X, preferred_element_type=jnp.float32)
    counts = jnp.maximum(jnp.sum(onehot, axis=1, keepdims=True), 1.0)
    g = gtok_ref[:, :] + sums / counts
    xg = _dot_t(g, wgin_ref[:, :]) + wginb_ref[:, :]
    x1 = X + jnp.dot(onehot_t, xg, preferred_element_type=jnp.float32)
    xn1 = _ln_rows(x1, n1w_ref[:, :], n1b_ref[:, :])
    x1_ref[:, :] = x1
    xn1_ref[:, :] = xn1
    m_ref[:, :] = _dot_t(xn1, msgw_ref[:, :]) + msgb_ref[:, :]


def _stage_a(X, batch_idx, g_token, Wg_in_W, Wg_in_b, norm1_w, norm1_b,
             local_msg_W, local_msg_b):
    out_shapes = (
        jax.ShapeDtypeStruct((N, D), jnp.float32),
        jax.ShapeDtypeStruct((N, D), jnp.float32),
        jax.ShapeDtypeStruct((N, D), jnp.float32),
    )
    return pl.pallas_call(_stage_a_body, out_shape=out_shapes)(
        X, batch_idx.reshape(1, N), batch_idx.reshape(N, 1),
        g_token.reshape(1, D), Wg_in_W, Wg_in_b.reshape(1, D),
        norm1_w.reshape(1, D), norm1_b.reshape(1, D),
        local_msg_W, local_msg_b.reshape(1, D))


@functools.partial(
    pl.kernel,
    out_type=jax.ShapeDtypeStruct((NC * N, D), jnp.float32),
    mesh=_sc_mesh,
    scratch_types=[
        pltpu.VMEM((C,), jnp.int32),
        pltpu.VMEM((C,), jnp.int32),
        pltpu.VMEM((C, D), jnp.float32),
        pltpu.VMEM((ZB, D), jnp.float32),
        pltpu.VMEM_SHARED((N, D), jnp.float32),
        pltpu.SemaphoreType.DMA,
    ],
    name="sc_mpnn_agg",
)
def _sc_agg(src_hbm, dst_hbm, m_hbm, out_hbm,
            src_v, dst_v, rows_v, zb_v, acc_sh, sem):
    cid = lax.axis_index("c")
    sid = lax.axis_index("s")
    base = (sid * NC + cid) * EPW
    _zero_bounce(zb_v)

    @pl.when(sid < NXS)
    def _():
        _init_acc(sid, zb_v, acc_sh)

    plsc.subcore_barrier()

    def chunk(i, carry):
        off = base + i * C
        pltpu.sync_copy(src_hbm.at[pl.ds(off, C)], src_v)
        pltpu.sync_copy(dst_hbm.at[pl.ds(off, C)], dst_v)
        pltpu.async_copy(m_hbm.at[dst_v], rows_v, sem).wait()
        pltpu.sync_copy(rows_v, acc_sh.at[src_v], add=True)
        return carry

    lax.fori_loop(0, NCHUNK, chunk, 0)
    plsc.subcore_barrier()

    @pl.when(sid < NXS)
    def _():
        _export_acc(cid, sid, zb_v, acc_sh, out_hbm)


def _stage_b_body(x1_ref, xn1_ref, agg0_ref, agg1_ref, eps_ref, linw_ref,
                  linb_ref, n2w_ref, n2b_ref, wq_ref, wk_ref, wy_ref,
                  wyb_ref, wphi_ref, x2_ref, qk_ref, vy_ref, wext_ref):
    xn1 = xn1_ref[:, :]
    agg = agg0_ref[:, :] + agg1_ref[:, :]
    pre = (1.0 + eps_ref[0, 0]) * xn1 + agg
    y = jnp.maximum(_dot_t(pre, linw_ref[:, :]) + linb_ref[:, :], 0.0)
    x2 = x1_ref[:, :] + y
    xn2 = _ln_rows(x2, n2w_ref[:, :], n2b_ref[:, :])
    qk_ref[:, 0:A] = _dot_t(xn2, wq_ref[:, :])
    qk_ref[:, A:2 * A] = _dot_t(xn2, wk_ref[:, :])
    vy_ref[:, :] = _dot_t(xn2, wy_ref[:, :]) + wyb_ref[:, :]
    x2_ref[:, :] = x2
    wphi = wphi_ref[:, :]
    wext_ref[:, :] = jnp.zeros((1, 80), jnp.float32)
    wext_ref[:, 0:A] = wphi
    wext_ref[:, A:A + 1] = jnp.sum(jnp.abs(wphi), axis=1, keepdims=True)


def _stage_b(X1, Xn1, aggp, local_eps, local_lin_W, local_lin_b,
             norm2_w, norm2_b, Wq, Wk, Wphi, Wy_W, Wy_b):
    out_shapes = (
        jax.ShapeDtypeStruct((N, D), jnp.float32),
        jax.ShapeDtypeStruct((N, 2 * A), jnp.float32),
        jax.ShapeDtypeStruct((N, D), jnp.float32),
        jax.ShapeDtypeStruct((1, 80), jnp.float32),
    )
    return pl.pallas_call(_stage_b_body, out_shape=out_shapes)(
        X1, Xn1, aggp[:N], aggp[N:], local_eps.reshape(1, 1),
        local_lin_W, local_lin_b.reshape(1, D), norm2_w.reshape(1, D),
        norm2_b.reshape(1, D), Wq, Wk, Wy_W, Wy_b.reshape(1, D), Wphi)


@functools.partial(
    pl.kernel,
    out_type=(
        jax.ShapeDtypeStruct((NC * N, D), jnp.float32),
        jax.ShapeDtypeStruct((E, A + 16), jnp.float32),
    ),
    mesh=_sc_mesh,
    scratch_types=[
        pltpu.VMEM((C,), jnp.int32),
        pltpu.VMEM((C,), jnp.int32),
        pltpu.VMEM((C, 2 * A), jnp.float32),
        pltpu.VMEM((C, 2 * A), jnp.float32),
        pltpu.VMEM((C, D), jnp.float32),
        pltpu.VMEM((C, A + 16), jnp.float32),
        pltpu.VMEM((ZB, D), jnp.float32),
        pltpu.VMEM((80,), jnp.float32),
        pltpu.VMEM((8, D), jnp.float32),
        pltpu.VMEM_SHARED((N, D), jnp.float32),
        pltpu.SemaphoreType.DMA,
    ],
    name="sc_attn_u",
)
def _sc_attu(src_hbm, dst_hbm, qk_hbm, vy_hbm, wext_hbm, dep_hbm,
             out_hbm, e_out_hbm,
             src_v, dst_v, qs_v, kd_v, vy_v, e_v, zb_v, w_v, dep_v,
             acc_sh, sem):
    cid = lax.axis_index("c")
    sid = lax.axis_index("s")
    base = (sid * NC + cid) * EPW
    pltpu.sync_copy(wext_hbm, w_v)
    pltpu.sync_copy(dep_hbm.at[pl.ds(0, 8)], dep_v)
    _zero_bounce(zb_v)

    @pl.when(sid < NXS)
    def _():
        _init_acc(sid, zb_v, acc_sh)

    plsc.subcore_barrier()
    wphi = [w_v[pl.ds(j * 16, 16)] for j in range(A // 16)]
    shift = w_v[pl.ds(A, 16)][0]

    def chunk(i, carry):
        off = base + i * C
        pltpu.sync_copy(src_hbm.at[pl.ds(off, C)], src_v)
        pltpu.sync_copy(dst_hbm.at[pl.ds(off, C)], dst_v)
        cp_q = pltpu.async_copy(qk_hbm.at[src_v], qs_v, sem)
        cp_k = pltpu.async_copy(qk_hbm.at[dst_v], kd_v, sem)
        cp_v = pltpu.async_copy(vy_hbm.at[dst_v], vy_v, sem)
        cp_q.wait()
        cp_k.wait()
        cp_v.wait()

        def edge(c, ecarry):
            acc = jnp.zeros((16,), jnp.float32)
            for j in range(A // 16):
                e = _tanh(qs_v[c, pl.ds(j * 16, 16)]
                          + kd_v[c, pl.ds(A + j * 16, 16)])
                e_v[c, pl.ds(j * 16, 16)] = e
                acc = acc + e * wphi[j]
            esv = jnp.exp(_allsum16(acc) - shift)
            e_v[c, pl.ds(A, 16)] = esv
            for j in range(D // 16):
                sl = pl.ds(j * 16, 16)
                vy_v[c, sl] = esv * vy_v[c, sl]
            return ecarry

        lax.fori_loop(0, C, edge, 0)
        pltpu.sync_copy(vy_v, acc_sh.at[src_v], add=True)
        pltpu.sync_copy(e_v, e_out_hbm.at[pl.ds(off, C)])
        return carry

    lax.fori_loop(0, NCHUNK, chunk, 0)
    plsc.subcore_barrier()

    @pl.when(sid < NXS)
    def _():
        _export_acc(cid, sid, zb_v, acc_sh, out_hbm)


@functools.partial(
    pl.kernel,
    out_type=jax.ShapeDtypeStruct((NC * N, D), jnp.float32),
    mesh=_sc_mesh,
    scratch_types=[
        pltpu.VMEM((C,), jnp.int32),
        pltpu.VMEM((C, A + 16), jnp.float32),
        pltpu.VMEM((C, D), jnp.float32),
        pltpu.VMEM((ZB, D), jnp.float32),
        pltpu.VMEM_SHARED((N, D), jnp.float32),
        pltpu.SemaphoreType.DMA,
    ],
    name="sc_attn_pe",
)
def _sc_attpe(src_hbm, e_hbm, out_hbm,
              src_v, e_v, pe_v, zb_v, acc_sh, sem):
    cid = lax.axis_index("c")
    sid = lax.axis_index("s")
    base = (sid * NC + cid) * EPW
    _zero_bounce(zb_v)

    @pl.when(sid < NXS)
    def _():
        _init_acc(sid, zb_v, acc_sh)

    plsc.subcore_barrier()
    lane0 = lax.broadcasted_iota(jnp.int32, (16,), 0) == 0
    zeros16 = jnp.zeros((16,), jnp.float32)

    def chunk(i, carry):
        off = base + i * C
        pltpu.sync_copy(src_hbm.at[pl.ds(off, C)], src_v)
        pltpu.sync_copy(e_hbm.at[pl.ds(off, C)], e_v)

        def edge(c, ecarry):
            esv = e_v[c, pl.ds(A, 16)]
            for j in range(A // 16):
                sl = pl.ds(j * 16, 16)
                pe_v[c, sl] = esv * e_v[c, sl]
            pe_v[c, pl.ds(A, 16)] = jnp.where(lane0, esv, 0.0)
            pe_v[c, pl.ds(A + 16, 16)] = zeros16
            pe_v[c, pl.ds(A + 32, 16)] = zeros16
            pe_v[c, pl.ds(A + 48, 16)] = zeros16
            return ecarry

        lax.fori_loop(0, C, edge, 0)
        pltpu.sync_copy(pe_v, acc_sh.at[src_v], add=True)
        return carry

    lax.fori_loop(0, NCHUNK, chunk, 0)
    plsc.subcore_barrier()

    @pl.when(sid < NXS)
    def _():
        _export_acc(cid, sid, zb_v, acc_sh, out_hbm)


def _stage_c_body(x2_ref, u0_ref, u1_ref, pe0_ref, pe1_ref,
                  wew_ref, web_ref, n3w_ref, n3b_ref, w1_ref, b1_ref,
                  w2_ref, b2_ref, out_ref):
    pe = pe0_ref[:, :] + pe1_ref[:, :]
    denom = pe[:, A:A + 1]
    r = 1.0 / (denom + 1e-9)
    p = pe[:, 0:A] * r
    u = (u0_ref[:, :] + u1_ref[:, :]) * r
    s = denom * r
    y2 = u + _dot_t(p, wew_ref[:, :]) + s * web_ref[:, :]
    x3 = x2_ref[:, :] + y2
    xn3 = _ln_rows(x3, n3w_ref[:, :], n3b_ref[:, :])
    h = jnp.maximum(_dot_t(xn3, w1_ref[:, :]) + b1_ref[:, :], 0.0)
    out_ref[:, :] = x3 + _dot_t(h, w2_ref[:, :]) + b2_ref[:, :]


def _stage_c(X2, Up, PEp, We_W, We_b, norm3_w, norm3_b,
             ffn1_W, ffn1_b, ffn2_W, ffn2_b):
    return pl.pallas_call(
        _stage_c_body,
        out_shape=jax.ShapeDtypeStruct((N, D), jnp.float32),
    )(X2, Up[:N], Up[N:], PEp[:N], PEp[N:], We_W, We_b.reshape(1, D),
      norm3_w.reshape(1, D), norm3_b.reshape(1, D),
      ffn1_W, ffn1_b.reshape(1, H), ffn2_W, ffn2_b.reshape(1, D))


def kernel(X, edge_index, batch_idx, g_token, Wg_in_W, Wg_in_b, norm1_w,
           norm1_b, local_eps, local_msg_W, local_msg_b, local_lin_W,
           local_lin_b, norm2_w, norm2_b, Wq, Wk, Wphi, Wy_W, Wy_b, We_W,
           We_b, norm3_w, norm3_b, ffn1_W, ffn1_b, ffn2_W, ffn2_b,
           Wg_out_W, Wg_out_b):
    src, dst = edge_index[0], edge_index[1]

    X1, Xn1, M = _stage_a(X, batch_idx, g_token, Wg_in_W, Wg_in_b,
                          norm1_w, norm1_b, local_msg_W, local_msg_b)
    aggp = _sc_agg(src, dst, M)
    X2, QK, VY, wext = _stage_b(X1, Xn1, aggp, local_eps, local_lin_W,
                                local_lin_b, norm2_w, norm2_b, Wq, Wk,
                                Wphi, Wy_W, Wy_b)
    wext = wext.reshape(80)
    Up, e_seq = _sc_attu(src, dst, QK, VY, wext, aggp)
    PEp = _sc_attpe(src, e_seq)
    return _stage_c(X2, Up, PEp, We_W, We_b, norm3_w, norm3_b,
                    ffn1_W, ffn1_b, ffn2_W, ffn2_b)


# speedup vs baseline: 117.5504x; 1.0015x over previous
import functools

import jax
import jax.numpy as jnp
from jax import lax
from jax.experimental import pallas as pl
from jax.experimental.pallas import tpu as pltpu
from jax.experimental.pallas import tpu_sc as plsc

N, E, D, A, H, NG = 10000, 320000, 128, 64, 256, 64

NC, NS = 2, 16
NW = NC * NS
EPW = E // NW
C = 80
NCHUNK = EPW // C
NXS = 10
NPS = N // NXS
ZB = 40
NZB = NPS // ZB

_sc_mesh = plsc.VectorSubcoreMesh(core_axis_name="c", subcore_axis_name="s")


def _dot_t(x, w):
    return lax.dot_general(x, w, (((1,), (1,)), ((), ())),
                           preferred_element_type=jnp.float32)


def _ln_rows(x, w, b):
    mu = jnp.mean(x, axis=-1, keepdims=True)
    var = jnp.mean((x - mu) ** 2, axis=-1, keepdims=True)
    return (x - mu) * lax.rsqrt(var + 1e-5) * w + b


def _tanh(x):
    t = jnp.exp(x * 2.0)
    return 1.0 - 2.0 / (t + 1.0)


def _allsum16(x):
    dnums = lax.GatherDimensionNumbers(
        offset_dims=(), collapsed_slice_dims=(0,), start_index_map=(0,))
    lanes = lax.broadcasted_iota(jnp.int32, (16,), 0)
    for k in (1, 2, 4, 8):
        perm = (lanes ^ k).reshape(16, 1)
        x = x + lax.gather(x, perm, dnums, slice_sizes=(1,),
                           mode=lax.GatherScatterMode.PROMISE_IN_BOUNDS)
    return x


def _zero_bounce(zb_v):
    def zrow(z, zcarry):
        for j in range(D // 16):
            zb_v[z, pl.ds(j * 16, 16)] = jnp.zeros((16,), jnp.float32)
        return zcarry

    lax.fori_loop(0, ZB, zrow, 0)


def _init_acc(sid, zb_v, acc_sh):
    def zcp(t, zcarry):
        pltpu.sync_copy(zb_v, acc_sh.at[pl.ds(sid * NPS + t * ZB, ZB)])
        return zcarry

    lax.fori_loop(0, NZB, zcp, 0)


def _export_acc(cid, sid, zb_v, acc_sh, out_hbm):
    def ecp(t, ecarry):
        roff = sid * NPS + t * ZB
        pltpu.sync_copy(acc_sh.at[pl.ds(roff, ZB)], zb_v)
        pltpu.sync_copy(zb_v, out_hbm.at[pl.ds(cid * N + roff, ZB)])
        return ecarry

    lax.fori_loop(0, NZB, ecp, 0)


def _stage_a_body(x_ref, bidx_row_ref, bidx_col_ref, gtok_ref, wgin_ref,
                  wginb_ref, n1w_ref, n1b_ref, msgw_ref, msgb_ref,
                  x1_ref, xn1_ref, m_ref):
    X = x_ref[:, :]
    iota_g = lax.broadcasted_iota(jnp.int32, (NG, N), 0)
    onehot = (bidx_row_ref[:, :] == iota_g).astype(jnp.float32)
    iota_n = lax.broadcasted_iota(jnp.int32, (N, NG), 1)
    onehot_t = (bidx_col_ref[:, :] == iota_n).astype(jnp.float32)
    sums = jnp.dot(onehot, X, preferred_element_type=jnp.float32)
    counts = jnp.maximum(jnp.sum(onehot, axis=1, keepdims=True), 1.0)
    g = gtok_ref[:, :] + sums / counts
    xg = _dot_t(g, wgin_ref[:, :]) + wginb_ref[:, :]
    x1 = X + jnp.dot(onehot_t, xg, preferred_element_type=jnp.float32)
    xn1 = _ln_rows(x1, n1w_ref[:, :], n1b_ref[:, :])
    x1_ref[:, :] = x1
    xn1_ref[:, :] = xn1
    m_ref[:, :] = _dot_t(xn1, msgw_ref[:, :]) + msgb_ref[:, :]


def _stage_a(X, batch_idx, g_token, Wg_in_W, Wg_in_b, norm1_w, norm1_b,
             local_msg_W, local_msg_b):
    out_shapes = (
        jax.ShapeDtypeStruct((N, D), jnp.float32),
        jax.ShapeDtypeStruct((N, D), jnp.float32),
        jax.ShapeDtypeStruct((N, D), jnp.float32),
    )
    return pl.pallas_call(_stage_a_body, out_shape=out_shapes)(
        X, batch_idx.reshape(1, N), batch_idx.reshape(N, 1),
        g_token.reshape(1, D), Wg_in_W, Wg_in_b.reshape(1, D),
        norm1_w.reshape(1, D), norm1_b.reshape(1, D),
        local_msg_W, local_msg_b.reshape(1, D))


@functools.partial(
    pl.kernel,
    out_type=jax.ShapeDtypeStruct((NC * N, D), jnp.float32),
    mesh=_sc_mesh,
    scratch_types=[
        pltpu.VMEM((C,), jnp.int32),
        pltpu.VMEM((C,), jnp.int32),
        pltpu.VMEM((C, D), jnp.float32),
        pltpu.VMEM((ZB, D), jnp.float32),
        pltpu.VMEM_SHARED((N, D), jnp.float32),
        pltpu.SemaphoreType.DMA,
    ],
    name="sc_mpnn_agg",
)
def _sc_agg(src_hbm, dst_hbm, m_hbm, out_hbm,
            src_v, dst_v, rows_v, zb_v, acc_sh, sem):
    cid = lax.axis_index("c")
    sid = lax.axis_index("s")
    base = (sid * NC + cid) * EPW
    _zero_bounce(zb_v)

    @pl.when(sid < NXS)
    def _():
        _init_acc(sid, zb_v, acc_sh)

    plsc.subcore_barrier()

    def chunk(i, carry):
        off = base + i * C
        pltpu.sync_copy(src_hbm.at[pl.ds(off, C)], src_v)
        pltpu.sync_copy(dst_hbm.at[pl.ds(off, C)], dst_v)
        pltpu.async_copy(m_hbm.at[dst_v], rows_v, sem).wait()
        pltpu.sync_copy(rows_v, acc_sh.at[src_v], add=True)
        return carry

    lax.fori_loop(0, NCHUNK, chunk, 0)
    plsc.subcore_barrier()

    @pl.when(sid < NXS)
    def _():
        _export_acc(cid, sid, zb_v, acc_sh, out_hbm)


def _stage_b_body(x1_ref, xn1_ref, agg0_ref, agg1_ref, eps_ref, linw_ref,
                  linb_ref, n2w_ref, n2b_ref, wq_ref, wk_ref, wy_ref,
                  wyb_ref, wphi_ref, x2_ref, qk_ref, vy_ref, wext_ref):
    xn1 = xn1_ref[:, :]
    agg = agg0_ref[:, :] + agg1_ref[:, :]
    pre = (1.0 + eps_ref[0, 0]) * xn1 + agg
    y = jnp.maximum(_dot_t(pre, linw_ref[:, :]) + linb_ref[:, :], 0.0)
    x2 = x1_ref[:, :] + y
    xn2 = _ln_rows(x2, n2w_ref[:, :], n2b_ref[:, :])
    qk_ref[:, 0:A] = _dot_t(xn2, wq_ref[:, :])
    qk_ref[:, A:2 * A] = _dot_t(xn2, wk_ref[:, :])
    vy_ref[:, :] = _dot_t(xn2, wy_ref[:, :]) + wyb_ref[:, :]
    x2_ref[:, :] = x2
    wphi = wphi_ref[:, :]
    wext_ref[:, :] = jnp.zeros((1, 80), jnp.float32)
    wext_ref[:, 0:A] = wphi
    wext_ref[:, A:A + 1] = jnp.sum(jnp.abs(wphi), axis=1, keepdims=True)


def _stage_b(X1, Xn1, aggp, local_eps, local_lin_W, local_lin_b,
             norm2_w, norm2_b, Wq, Wk, Wphi, Wy_W, Wy_b):
    out_shapes = (
        jax.ShapeDtypeStruct((N, D), jnp.float32),
        jax.ShapeDtypeStruct((N, 2 * A), jnp.float32),
        jax.ShapeDtypeStruct((N, D), jnp.float32),
        jax.ShapeDtypeStruct((1, 80), jnp.float32),
    )
    return pl.pallas_call(_stage_b_body, out_shape=out_shapes)(
        X1, Xn1, aggp[:N], aggp[N:], local_eps.reshape(1, 1),
        local_lin_W, local_lin_b.reshape(1, D), norm2_w.reshape(1, D),
        norm2_b.reshape(1, D), Wq, Wk, Wy_W, Wy_b.reshape(1, D), Wphi)


@functools.partial(
    pl.kernel,
    out_type=(
        jax.ShapeDtypeStruct((NC * N, D), jnp.float32),
        jax.ShapeDtypeStruct((E, A + 16), jnp.float32),
    ),
    mesh=_sc_mesh,
    scratch_types=[
        pltpu.VMEM((C,), jnp.int32),
        pltpu.VMEM((C,), jnp.int32),
        pltpu.VMEM((C, 2 * A), jnp.float32),
        pltpu.VMEM((C, 2 * A), jnp.float32),
        pltpu.VMEM((C, D), jnp.float32),
        pltpu.VMEM((C, A + 16), jnp.float32),
        pltpu.VMEM((ZB, D), jnp.float32),
        pltpu.VMEM((80,), jnp.float32),
        pltpu.VMEM((8, D), jnp.float32),
        pltpu.VMEM_SHARED((N, D), jnp.float32),
        pltpu.SemaphoreType.DMA,
    ],
    name="sc_attn_u",
)
def _sc_attu(src_hbm, dst_hbm, qk_hbm, vy_hbm, wext_hbm, dep_hbm,
             out_hbm, e_out_hbm,
             src_v, dst_v, qs_v, kd_v, vy_v, e_v, zb_v, w_v, dep_v,
             acc_sh, sem):
    cid = lax.axis_index("c")
    sid = lax.axis_index("s")
    base = (sid * NC + cid) * EPW
    pltpu.sync_copy(wext_hbm, w_v)
    pltpu.sync_copy(dep_hbm.at[pl.ds(0, 8)], dep_v)
    _zero_bounce(zb_v)

    @pl.when(sid < NXS)
    def _():
        _init_acc(sid, zb_v, acc_sh)

    plsc.subcore_barrier()
    wphi = [w_v[pl.ds(j * 16, 16)] for j in range(A // 16)]
    shift = w_v[pl.ds(A, 16)][0]

    def chunk(i, carry):
        off = base + i * C
        pltpu.sync_copy(src_hbm.at[pl.ds(off, C)], src_v)
        pltpu.sync_copy(dst_hbm.at[pl.ds(off, C)], dst_v)
        cp_q = pltpu.async_copy(qk_hbm.at[src_v], qs_v, sem)
        cp_k = pltpu.async_copy(qk_hbm.at[dst_v], kd_v, sem)
        cp_v = pltpu.async_copy(vy_hbm.at[dst_v], vy_v, sem)
        cp_q.wait()
        cp_k.wait()
        cp_v.wait()

        def edge(c2, ecarry):
            for dc in range(2):
                c = c2 * 2 + dc
                acc = jnp.zeros((16,), jnp.float32)
                for j in range(A // 16):
                    e = _tanh(qs_v[c, pl.ds(j * 16, 16)]
                              + kd_v[c, pl.ds(A + j * 16, 16)])
                    e_v[c, pl.ds(j * 16, 16)] = e
                    acc = acc + e * wphi[j]
                esv = jnp.exp(_allsum16(acc) - shift)
                e_v[c, pl.ds(A, 16)] = esv
                for j in range(D // 16):
                    sl = pl.ds(j * 16, 16)
                    vy_v[c, sl] = esv * vy_v[c, sl]
            return ecarry

        lax.fori_loop(0, C // 2, edge, 0)
        pltpu.sync_copy(vy_v, acc_sh.at[src_v], add=True)
        pltpu.sync_copy(e_v, e_out_hbm.at[pl.ds(off, C)])
        return carry

    lax.fori_loop(0, NCHUNK, chunk, 0)
    plsc.subcore_barrier()

    @pl.when(sid < NXS)
    def _():
        _export_acc(cid, sid, zb_v, acc_sh, out_hbm)


@functools.partial(
    pl.kernel,
    out_type=jax.ShapeDtypeStruct((NC * N, D), jnp.float32),
    mesh=_sc_mesh,
    scratch_types=[
        pltpu.VMEM((C,), jnp.int32),
        pltpu.VMEM((C, A + 16), jnp.float32),
        pltpu.VMEM((C, D), jnp.float32),
        pltpu.VMEM((ZB, D), jnp.float32),
        pltpu.VMEM_SHARED((N, D), jnp.float32),
        pltpu.SemaphoreType.DMA,
    ],
    name="sc_attn_pe",
)
def _sc_attpe(src_hbm, e_hbm, out_hbm,
              src_v, e_v, pe_v, zb_v, acc_sh, sem):
    cid = lax.axis_index("c")
    sid = lax.axis_index("s")
    base = (sid * NC + cid) * EPW
    _zero_bounce(zb_v)

    @pl.when(sid < NXS)
    def _():
        _init_acc(sid, zb_v, acc_sh)

    plsc.subcore_barrier()
    lane0 = lax.broadcasted_iota(jnp.int32, (16,), 0) == 0
    zeros16 = jnp.zeros((16,), jnp.float32)

    def chunk(i, carry):
        off = base + i * C
        pltpu.sync_copy(src_hbm.at[pl.ds(off, C)], src_v)
        pltpu.sync_copy(e_hbm.at[pl.ds(off, C)], e_v)

        def edge(c2, ecarry):
            for dc in range(2):
                c = c2 * 2 + dc
                esv = e_v[c, pl.ds(A, 16)]
                for j in range(A // 16):
                    sl = pl.ds(j * 16, 16)
                    pe_v[c, sl] = esv * e_v[c, sl]
                pe_v[c, pl.ds(A, 16)] = jnp.where(lane0, esv, 0.0)
                pe_v[c, pl.ds(A + 16, 16)] = zeros16
                pe_v[c, pl.ds(A + 32, 16)] = zeros16
                pe_v[c, pl.ds(A + 48, 16)] = zeros16
            return ecarry

        lax.fori_loop(0, C // 2, edge, 0)
        pltpu.sync_copy(pe_v, acc_sh.at[src_v], add=True)
        return carry

    lax.fori_loop(0, NCHUNK, chunk, 0)
    plsc.subcore_barrier()

    @pl.when(sid < NXS)
    def _():
        _export_acc(cid, sid, zb_v, acc_sh, out_hbm)


def _stage_c_body(x2_ref, u0_ref, u1_ref, pe0_ref, pe1_ref,
                  wew_ref, web_ref, n3w_ref, n3b_ref, w1_ref, b1_ref,
                  w2_ref, b2_ref, out_ref):
    pe = pe0_ref[:, :] + pe1_ref[:, :]
    denom = pe[:, A:A + 1]
    r = 1.0 / (denom + 1e-9)
    p = pe[:, 0:A] * r
    u = (u0_ref[:, :] + u1_ref[:, :]) * r
    s = denom * r
    y2 = u + _dot_t(p, wew_ref[:, :]) + s * web_ref[:, :]
    x3 = x2_ref[:, :] + y2
    xn3 = _ln_rows(x3, n3w_ref[:, :], n3b_ref[:, :])
    h = jnp.maximum(_dot_t(xn3, w1_ref[:, :]) + b1_ref[:, :], 0.0)
    out_ref[:, :] = x3 + _dot_t(h, w2_ref[:, :]) + b2_ref[:, :]


def _stage_c(X2, Up, PEp, We_W, We_b, norm3_w, norm3_b,
             ffn1_W, ffn1_b, ffn2_W, ffn2_b):
    return pl.pallas_call(
        _stage_c_body,
        out_shape=jax.ShapeDtypeStruct((N, D), jnp.float32),
    )(X2, Up[:N], Up[N:], PEp[:N], PEp[N:], We_W, We_b.reshape(1, D),
      norm3_w.reshape(1, D), norm3_b.reshape(1, D),
      ffn1_W, ffn1_b.reshape(1, H), ffn2_W, ffn2_b.reshape(1, D))


def kernel(X, edge_index, batch_idx, g_token, Wg_in_W, Wg_in_b, norm1_w,
           norm1_b, local_eps, local_msg_W, local_msg_b, local_lin_W,
           local_lin_b, norm2_w, norm2_b, Wq, Wk, Wphi, Wy_W, Wy_b, We_W,
           We_b, norm3_w, norm3_b, ffn1_W, ffn1_b, ffn2_W, ffn2_b,
           Wg_out_W, Wg_out_b):
    src, dst = edge_index[0], edge_index[1]

    X1, Xn1, M = _stage_a(X, batch_idx, g_token, Wg_in_W, Wg_in_b,
                          norm1_w, norm1_b, local_msg_W, local_msg_b)
    aggp = _sc_agg(src, dst, M)
    X2, QK, VY, wext = _stage_b(X1, Xn1, aggp, local_eps, local_lin_W,
                                local_lin_b, norm2_w, norm2_b, Wq, Wk,
                                Wphi, Wy_W, Wy_b)
    wext = wext.reshape(80)
    Up, e_seq = _sc_attu(src, dst, QK, VY, wext, aggp)
    PEp = _sc_attpe(src, e_seq)
    return _stage_c(X2, Up, PEp, We_W, We_b, norm3_w, norm3_b,
                    ffn1_W, ffn1_b, ffn2_W, ffn2_b)
